# baseline (speedup 1.0000x reference)
import functools

import jax
import jax.numpy as jnp
from jax import lax
from jax.experimental import pallas as pl
from jax.experimental.pallas import tpu as pltpu
from jax.experimental.pallas import tpu_sc as plsc

N_NODES = 10000
N_EDGES = 320000
D = 128

CHUNK = 128
NCORES = 2
NSUB = 16
NWORKERS = NCORES * NSUB
WCHUNKS = 80
PCHUNKS = NWORKERS * WCHUNKS
PAD_EDGES = PCHUNKS * CHUNK
SCHUNKS = 16
NSTAGES = WCHUNKS // SCHUNKS
RCHUNK = 80
NRCHUNKS = N_NODES // RCHUNK
RITERS = -(-NRCHUNKS // NSUB)


def _mm_body(x_ref, w_ref, o_ref):
    o_ref[...] = jnp.dot(x_ref[...], w_ref[...],
                         preferred_element_type=jnp.float32)


def _matmul(x, w):
    return pl.pallas_call(
        _mm_body,
        grid=(10,),
        in_specs=[
            pl.BlockSpec((N_NODES // 10, D), lambda i: (i, 0)),
            pl.BlockSpec((D, D), lambda i: (0, 0)),
        ],
        out_specs=pl.BlockSpec((N_NODES // 10, D), lambda i: (i, 0)),
        out_shape=jax.ShapeDtypeStruct((N_NODES, D), jnp.float32),
    )(x, w)


def _comb_body(p_ref, b_ref, o_ref):
    o_ref[...] = p_ref[0] + p_ref[1] + b_ref[...]


def _combine(partials, b):
    return pl.pallas_call(
        _comb_body,
        grid=(10,),
        in_specs=[
            pl.BlockSpec((2, N_NODES // 10, D), lambda i: (0, i, 0)),
            pl.BlockSpec((1, D), lambda i: (0, 0)),
        ],
        out_specs=pl.BlockSpec((N_NODES // 10, D), lambda i: (i, 0)),
        out_shape=jax.ShapeDtypeStruct((N_NODES, D), jnp.float32),
    )(partials, b)


def _scale_rows(rows_ref, vals_blk, c):
    @pl.loop(0, CHUNK // 16)
    def _(eb):
        vals16 = vals_blk[pl.ds(c, 1), pl.ds(eb * 16, 16)]
        for j in range(16):
            v = vals16[0, j]
            for g in range(D // 16):
                sl = (pl.ds(eb * 16 + j, 1), pl.ds(g * 16, 16))
                rows_ref[sl] = rows_ref[sl] * v


def _spmm(hidden, src, dst, vals):
    mesh = plsc.VectorSubcoreMesh(core_axis_name="core",
                                  subcore_axis_name="subcore")

    @functools.partial(
        pl.kernel,
        out_type=jax.ShapeDtypeStruct((NCORES, N_NODES, D), jnp.float32),
        mesh=mesh,
        scratch_types=[
            pltpu.VMEM((SCHUNKS, CHUNK), jnp.int32),
            pltpu.VMEM((SCHUNKS, CHUNK), jnp.int32),
            pltpu.VMEM((SCHUNKS, CHUNK), jnp.float32),
            pltpu.VMEM((CHUNK, D), jnp.float32),
            pltpu.VMEM((CHUNK, D), jnp.float32),
            pltpu.VMEM_SHARED((N_NODES, D), jnp.float32),
        ],
    )
    def spmm_kernel(hid_hbm, src_hbm, dst_hbm, val_hbm, part_hbm,
                    sidx_v, didx_v, val_v, rows0, rows1, acc):
        cid = lax.axis_index("core")
        tid = lax.axis_index("subcore")
        wid = tid * NCORES + cid
        base = wid * WCHUNKS

        @pl.loop(0, RCHUNK)
        def _(r):
            for g in range(D // 16):
                rows0[pl.ds(r, 1), pl.ds(g * 16, 16)] = jnp.zeros(
                    (1, 16), jnp.float32)

        zsrc = rows0.at[pl.ds(0, RCHUNK)]
        for k in range(RITERS):
            rc = k * NSUB + tid

            @pl.when(rc < NRCHUNKS)
            def _():
                pltpu.sync_copy(zsrc, acc.at[pl.ds(rc * RCHUNK, RCHUNK)])
        plsc.subcore_barrier()

        @pl.loop(0, NSTAGES)
        def _(s):
            sbase = base + s * SCHUNKS
            pltpu.sync_copy(src_hbm.at[pl.ds(sbase, SCHUNKS)], sidx_v)
            pltpu.sync_copy(dst_hbm.at[pl.ds(sbase, SCHUNKS)], didx_v)
            pltpu.sync_copy(val_hbm.at[pl.ds(sbase, SCHUNKS)], val_v)

            @pl.loop(0, SCHUNKS)
            def _(cc):
                pltpu.sync_copy(hid_hbm.at[sidx_v.at[cc]], rows0)
                _scale_rows(rows0, val_v, cc)
                pltpu.sync_copy(rows0, acc.at[didx_v.at[cc]], add=True)

        plsc.subcore_barrier()

        for k in range(RITERS):
            rc = k * NSUB + tid

            @pl.when(rc < NRCHUNKS)
            def _():
                pltpu.sync_copy(
                    acc.at[pl.ds(rc * RCHUNK, RCHUNK)],
                    part_hbm.at[cid, pl.ds(rc * RCHUNK, RCHUNK)])

    return spmm_kernel(hidden, src, dst, vals)


def _pad_chunks(a, dtype):
    a = a.astype(dtype)
    pad = jnp.zeros((PAD_EDGES - N_EDGES,), dtype)
    return jnp.concatenate([a, pad]).reshape(PCHUNKS, CHUNK)


def kernel(input, edge_index, edge_vals, W, b):
    hidden = _matmul(input, W)
    dst = _pad_chunks(edge_index[0], jnp.int32)
    src = _pad_chunks(edge_index[1], jnp.int32)
    vals = _pad_chunks(edge_vals, jnp.float32)
    partials = _spmm(hidden, src, dst, vals)
    return _combine(partials, b)


# speedup vs baseline: 2.0973x; 2.0973x over previous
import dataclasses
import functools

import jax
import jax.numpy as jnp
from jax import lax
from jax.experimental import pallas as pl
from jax.experimental.pallas import tpu as pltpu
from jax.experimental.pallas import tpu_sc as plsc

N_NODES = 10000
N_EDGES = 320000
D = 128

CHUNK = 128
NCHUNKS = N_EDGES // CHUNK
NCORES = 2
NSUB = 16
NWORKERS = NCORES * NSUB
ITERS = -(-NCHUNKS // NWORKERS)
PAIRS = (ITERS + 1) // 2
RCHUNK = 80
NRCHUNKS = N_NODES // RCHUNK
RITERS = -(-NRCHUNKS // NSUB)


def _mm_body(x_ref, w_ref, o_ref):
    o_ref[...] = jnp.dot(x_ref[...], w_ref[...],
                         preferred_element_type=jnp.float32)


def _matmul(x, w):
    return pl.pallas_call(
        _mm_body,
        grid=(10,),
        in_specs=[
            pl.BlockSpec((N_NODES // 10, D), lambda i: (i, 0)),
            pl.BlockSpec((D, D), lambda i: (0, 0)),
        ],
        out_specs=pl.BlockSpec((N_NODES // 10, D), lambda i: (i, 0)),
        out_shape=jax.ShapeDtypeStruct((N_NODES, D), jnp.float32),
    )(x, w)


def _comb_body(p_ref, b_ref, o_ref):
    o_ref[...] = p_ref[0] + p_ref[1] + b_ref[...]


def _combine(partials, b):
    return pl.pallas_call(
        _comb_body,
        grid=(10,),
        in_specs=[
            pl.BlockSpec((2, N_NODES // 10, D), lambda i: (0, i, 0)),
            pl.BlockSpec((1, D), lambda i: (0, 0)),
        ],
        out_specs=pl.BlockSpec((N_NODES // 10, D), lambda i: (i, 0)),
        out_shape=jax.ShapeDtypeStruct((N_NODES, D), jnp.float32),
    )(partials, b)


def _scale_rows(rows_ref, vbuf):
    @pl.loop(0, CHUNK // 16)
    def _(eb):
        v16 = vbuf[pl.ds(0, 1), pl.ds(eb * 16, 16)]
        for j in range(16):
            v = v16[0, j]
            for g in range(D // 16):
                sl = (pl.ds(eb * 16 + j, 1), pl.ds(g * 16, 16))
                rows_ref[sl] = rows_ref[sl] * v


def _spmm(hidden, eidx, vals):
    mesh = plsc.VectorSubcoreMesh(core_axis_name="core",
                                  subcore_axis_name="subcore")

    @functools.partial(
        pl.kernel,
        out_type=jax.ShapeDtypeStruct((NCORES, N_NODES, D), jnp.float32),
        mesh=mesh,
        scratch_types=[
            pltpu.VMEM((2, CHUNK), jnp.int32),
            pltpu.VMEM((2, CHUNK), jnp.int32),
            pltpu.VMEM((1, CHUNK), jnp.float32),
            pltpu.VMEM((1, CHUNK), jnp.float32),
            pltpu.VMEM((CHUNK, D), jnp.float32),
            pltpu.VMEM((CHUNK, D), jnp.float32),
            pltpu.VMEM_SHARED((N_NODES, D), jnp.float32),
            pltpu.SemaphoreType.DMA,
            pltpu.SemaphoreType.DMA,
        ],
    )
    def spmm_kernel(hid_hbm, eidx_hbm, val_hbm, part_hbm,
                    ebuf0, ebuf1, vbuf0, vbuf1, rows0, rows1, acc,
                    ssem0, ssem1):
        cid = lax.axis_index("core")
        tid = lax.axis_index("subcore")
        wid = tid * NCORES + cid

        @pl.loop(0, RCHUNK)
        def _(r):
            for g in range(D // 16):
                rows0[pl.ds(r, 1), pl.ds(g * 16, 16)] = jnp.zeros(
                    (1, 16), jnp.float32)

        zsrc = rows0.at[pl.ds(0, RCHUNK)]
        for k in range(RITERS):
            rc = k * NSUB + tid

            @pl.when(rc < NRCHUNKS)
            def _():
                pltpu.sync_copy(zsrc, acc.at[pl.ds(rc * RCHUNK, RCHUNK)])
        plsc.subcore_barrier()

        @pl.loop(0, PAIRS)
        def _(h):
            for p, (ebuf, vbuf, rows_v, ssem) in enumerate(
                    ((ebuf0, vbuf0, rows0, ssem0),
                     (ebuf1, vbuf1, rows1, ssem1))):
                i = h * 2 + p
                chunk = i * NWORKERS + wid

                @pl.when(chunk < NCHUNKS)
                def _():
                    @pl.when(i >= 2)
                    def _():
                        pltpu.make_async_copy(
                            rows_v, acc.at[ebuf.at[1]], ssem).wait()

                    pltpu.sync_copy(eidx_hbm.at[chunk], ebuf)
                    pltpu.sync_copy(val_hbm.at[pl.ds(chunk, 1)], vbuf)
                    pltpu.sync_copy(hid_hbm.at[ebuf.at[0]], rows_v)
                    _scale_rows(rows_v, vbuf)
                    pltpu.async_copy(
                        rows_v, acc.at[ebuf.at[1]], ssem, add=True)

        pltpu.make_async_copy(rows0, acc.at[ebuf0.at[1]], ssem0).wait()
        pltpu.make_async_copy(rows1, acc.at[ebuf1.at[1]], ssem1).wait()
        plsc.subcore_barrier()

        for k in range(RITERS):
            rc = k * NSUB + tid

            @pl.when(rc < NRCHUNKS)
            def _():
                pltpu.sync_copy(
                    acc.at[pl.ds(rc * RCHUNK, RCHUNK)],
                    part_hbm.at[cid, pl.ds(rc * RCHUNK, RCHUNK)])

    return spmm_kernel(hidden, eidx, vals)


def kernel(input, edge_index, edge_vals, W, b):
    hidden = _matmul(input, W)
    dst = edge_index[0].astype(jnp.int32).reshape(NCHUNKS, CHUNK)
    src = edge_index[1].astype(jnp.int32).reshape(NCHUNKS, CHUNK)
    eidx = jnp.stack([src, dst], axis=1)
    vals = edge_vals.astype(jnp.float32).reshape(NCHUNKS, CHUNK)
    partials = _spmm(hidden, eidx, vals)
    return _combine(partials, b)
